# baseline (speedup 1.0000x reference)
import math

import jax
import jax.numpy as jnp
from jax.experimental import pallas as pl
import jax.experimental.pallas.tpu as pltpu

H = 768
V = 32000
NH = 12
DH = H // NH
E = 8
FF = 1536
S = 2048
EPS = 1e-5


def _ln(x, g=None, b=None):
    m = jnp.mean(x, axis=-1, keepdims=True)
    v = jnp.mean(x * x, axis=-1, keepdims=True) - m * m
    y = (x - m) * jax.lax.rsqrt(v + EPS)
    if g is not None:
        y = y * g + b
    return y


def _dot(a, b):
    return jnp.dot(a.astype(jnp.bfloat16), b.astype(jnp.bfloat16),
                   preferred_element_type=jnp.float32)


def _pre_body(hs_ref, te_ref, pw1_ref, pw2_ref, pb_ref,
              n1g_ref, n1b_ref, ob_ref, xn_ref, y_ref):
    c0 = (_dot(_ln(hs_ref[...]), pw1_ref[...])
          + _dot(_ln(te_ref[...]), pw2_ref[...]) + pb_ref[...])
    xn_ref[...] = _ln(c0, n1g_ref[...], n1b_ref[...]).astype(jnp.bfloat16)
    y_ref[...] = c0 + ob_ref[...]


def _prologue(hs, te, pw1, pw2, pb, n1g, n1b, ob):
    return pl.pallas_call(
        _pre_body,
        out_shape=(jax.ShapeDtypeStruct((S, H), jnp.bfloat16),
                   jax.ShapeDtypeStruct((S, H), jnp.float32)),
    )(hs, te, pw1, pw2, pb, n1g, n1b, ob)


def _attn_body(xn_ref, y0_ref, wq_ref, wk_ref, wv_ref,
               bq_ref, bk_ref, bv_ref, wo_ref, out_ref):
    h = pl.program_id(0)
    xn = xn_ref[...]
    q = _dot(xn, wq_ref[0]) + bq_ref[0]
    k = _dot(xn, wk_ref[0]) + bk_ref[0]
    v = _dot(xn, wv_ref[0]) + bv_ref[0]
    sc = jax.lax.dot_general(q.astype(jnp.bfloat16), k.astype(jnp.bfloat16),
                             (((1,), (1,)), ((), ())),
                             preferred_element_type=jnp.float32)
    sc = sc * (1.0 / math.sqrt(DH))
    sc = sc - jnp.max(sc, axis=-1, keepdims=True)
    p = jnp.exp(sc)
    r = jnp.sum(p, axis=-1, keepdims=True)
    o = _dot(p, v) / r
    contrib = _dot(o, wo_ref[...])

    @pl.when(h == 0)
    def _():
        out_ref[...] = y0_ref[...] + contrib

    @pl.when(h > 0)
    def _():
        out_ref[...] += contrib


def _attention(xn, y0, qkv_Ws, qkv_bs, out_W):
    const = lambda h: (0, 0)
    specs = [
        pl.BlockSpec((S, H), const),
        pl.BlockSpec((S, H), const),
        pl.BlockSpec((1, H, DH), lambda h: (h, 0, 0)),
        pl.BlockSpec((1, H, DH), lambda h: (NH + h, 0, 0)),
        pl.BlockSpec((1, H, DH), lambda h: (2 * NH + h, 0, 0)),
        pl.BlockSpec((1, 1, DH), lambda h: (h, 0, 0)),
        pl.BlockSpec((1, 1, DH), lambda h: (NH + h, 0, 0)),
        pl.BlockSpec((1, 1, DH), lambda h: (2 * NH + h, 0, 0)),
        pl.BlockSpec((DH, H), lambda h: (h, 0)),
    ]
    return pl.pallas_call(
        _attn_body,
        grid=(NH,),
        in_specs=specs,
        out_specs=pl.BlockSpec((S, H), const),
        out_shape=jax.ShapeDtypeStruct((S, H), jnp.float32),
    )(xn, y0, qkv_Ws, qkv_Ws, qkv_Ws, qkv_bs, qkv_bs, qkv_bs, out_W)


def _moe_body(y_ref, g_ref, b_ref, gw_ref, gb_ref,
              w1_ref, b1_ref, w2_ref, b2_ref, out_ref, x2_s, wv_s):
    e = pl.program_id(0)

    @pl.when(e == 0)
    def _gate():
        x2 = _ln(y_ref[...], g_ref[...], b_ref[...])
        x2_s[...] = x2
        logits = _dot(x2, gw_ref[...]) + gb_ref[...]
        lane = jax.lax.broadcasted_iota(jnp.int32, logits.shape, 1)
        logits = jnp.where(lane < E, logits, -1e30)
        logits = logits - jnp.max(logits, axis=-1, keepdims=True)
        pexp = jnp.exp(logits)
        probs = pexp / jnp.sum(pexp, axis=-1, keepdims=True)
        m1 = jnp.max(probs, axis=-1, keepdims=True)
        m2 = jnp.max(jnp.where(probs == m1, -1.0, probs),
                     axis=-1, keepdims=True)
        wv_s[...] = jnp.where(probs >= m2, probs, 0.0) / (m1 + m2)

    x2 = x2_s[...]
    lane = jax.lax.broadcasted_iota(jnp.int32, (S, 128), 1)
    onehot = (lane == e).astype(jnp.float32)
    we = jnp.sum(wv_s[...] * onehot, axis=-1, keepdims=True)
    hmat = jnp.maximum(_dot(x2, w1_ref[0]) + b1_ref[0], 0.0)
    contrib = (_dot(hmat, w2_ref[0]) + b2_ref[0]) * we

    @pl.when(e == 0)
    def _():
        out_ref[...] = y_ref[...] + contrib

    @pl.when(e > 0)
    def _():
        out_ref[...] += contrib


def _moe(y, n2g, n2b, gw_pad, gb_pad, w1, b1, w2, b2):
    const = lambda e: (0, 0)
    specs = [
        pl.BlockSpec((S, H), const),
        pl.BlockSpec((1, H), const),
        pl.BlockSpec((1, H), const),
        pl.BlockSpec((H, 128), const),
        pl.BlockSpec((1, 128), const),
        pl.BlockSpec((1, H, FF), lambda e: (e, 0, 0)),
        pl.BlockSpec((1, 1, FF), lambda e: (e, 0, 0)),
        pl.BlockSpec((1, FF, H), lambda e: (e, 0, 0)),
        pl.BlockSpec((1, 1, H), lambda e: (e, 0, 0)),
    ]
    return pl.pallas_call(
        _moe_body,
        grid=(E,),
        in_specs=specs,
        out_specs=pl.BlockSpec((S, H), const),
        out_shape=jax.ShapeDtypeStruct((S, H), jnp.float32),
        scratch_shapes=[
            pltpu.VMEM((S, H), jnp.float32),
            pltpu.VMEM((S, 128), jnp.float32),
        ],
    )(y, n2g, n2b, gw_pad, gb_pad, w1, b1, w2, b2)


VB2 = 512
N_CH = NH + 2 * E
N_HB = (V + VB2 - 1) // VB2
N_TRIPLE = 3 * N_CH
N_STEPS = N_TRIPLE + (N_HB - 2 * N_CH)
QC = 256
TC = 1024


def _mb_c(s):
    return jnp.minimum(s // 3, N_CH - 1)


def _mb_a(s):
    return jnp.clip(_mb_c(s), 0, NH - 1)


def _mb_m(s):
    return jnp.clip(_mb_c(s) - NH, 0, 2 * E - 1)


def _mb_j(s):
    return jnp.where(s < N_TRIPLE,
                     jnp.maximum(2 * (s // 3) + (s % 3) - 1, 0),
                     2 * N_CH + (s - N_TRIPLE))


def _megab_body(xn_ref, y0_ref, wq_ref, wk_ref, wv_ref,
                bq_ref, bk_ref, bv_ref, wo_ref,
                n2g_ref, n2b_ref, gw_ref, gb_ref,
                w1_ref, b1_ref, w2_ref, b2_ref,
                x0_ref, hw_ref, hb_ref,
                z_ref, out0_ref, xs, wv_s):
    s = pl.program_id(0)
    is_chain = (s < N_TRIPLE) & (s % 3 == 0)
    cs = _mb_c(s)

    @pl.when(is_chain & (cs < NH))
    def _att():
        xn = xn_ref[...]
        q = _dot(xn, wq_ref[0]) + bq_ref[0]
        k = _dot(xn, wk_ref[0]) + bk_ref[0]
        v = _dot(xn, wv_ref[0]) + bv_ref[0]
        kb = k.astype(jnp.bfloat16)
        for i in range(S // QC):
            qc = q[i * QC:(i + 1) * QC, :]
            sc = jax.lax.dot_general(qc.astype(jnp.bfloat16), kb,
                                     (((1,), (1,)), ((), ())),
                                     preferred_element_type=jnp.float32)
            sc = sc * (1.0 / math.sqrt(DH))
            sc = sc - jnp.max(sc, axis=-1, keepdims=True)
            p = jnp.exp(sc)
            r = jnp.sum(p, axis=-1, keepdims=True)
            o = _dot(p, v) / r
            contrib = _dot(o, wo_ref[...])

            @pl.when(cs == 0)
            def _():
                z_ref[i * QC:(i + 1) * QC, :] = (
                    y0_ref[i * QC:(i + 1) * QC, :] + contrib)

            @pl.when(cs > 0)
            def _():
                z_ref[i * QC:(i + 1) * QC, :] += contrib

    @pl.when(is_chain & (cs >= NH))
    def _moe():
        mm = cs - NH
        e = mm // 2
        f = mm % 2

        @pl.when(mm == 0)
        def _gate():
            x2 = _ln(z_ref[...], n2g_ref[...], n2b_ref[...])
            xs[...] = x2
            logits = _dot(x2, gw_ref[...]) + gb_ref[...]
            lane = jax.lax.broadcasted_iota(jnp.int32, logits.shape, 1)
            logits = jnp.where(lane < E, logits, -1e30)
            logits = logits - jnp.max(logits, axis=-1, keepdims=True)
            pexp = jnp.exp(logits)
            probs = pexp / jnp.sum(pexp, axis=-1, keepdims=True)
            m1 = jnp.max(probs, axis=-1, keepdims=True)
            m2 = jnp.max(jnp.where(probs == m1, -1.0, probs),
                         axis=-1, keepdims=True)
            wv_s[...] = jnp.where(probs >= m2, probs, 0.0) / (m1 + m2)

        lane = jax.lax.broadcasted_iota(jnp.int32, (TC, 128), 1)
        fb2 = (f == 0).astype(jnp.float32) * b2_ref[0]
        for t in range(S // TC):
            sl = slice(t * TC, (t + 1) * TC)
            x2 = xs[sl, :]
            onehot = (lane == e).astype(jnp.float32)
            we = jnp.sum(wv_s[sl, :] * onehot, axis=-1, keepdims=True)
            hmat = jnp.maximum(_dot(x2, w1_ref[0]) + b1_ref[0], 0.0)
            eo = _dot(hmat, w2_ref[0]) + fb2
            z_ref[sl, :] += eo * we

    @pl.when(jnp.logical_not(is_chain))
    def _head():
        out0_ref[...] = _dot(x0_ref[...], hw_ref[...]) + hb_ref[...]


def _megab(xn, y0, qkv_Ws, qkv_bs, out_W, n2g, n2b, gw_pad, gb_pad,
           w1, b1, w2, b2, x0, hw0, hb0):
    const = lambda s: (0, 0)
    a, m, j = _mb_a, _mb_m, _mb_j
    specs = [
        pl.BlockSpec((S, H), const),
        pl.BlockSpec((S, H), const),
        pl.BlockSpec((1, H, DH), lambda s: (a(s), 0, 0)),
        pl.BlockSpec((1, H, DH), lambda s: (NH + a(s), 0, 0)),
        pl.BlockSpec((1, H, DH), lambda s: (2 * NH + a(s), 0, 0)),
        pl.BlockSpec((1, 1, DH), lambda s: (a(s), 0, 0)),
        pl.BlockSpec((1, 1, DH), lambda s: (NH + a(s), 0, 0)),
        pl.BlockSpec((1, 1, DH), lambda s: (2 * NH + a(s), 0, 0)),
        pl.BlockSpec((DH, H), lambda s: (a(s), 0)),
        pl.BlockSpec((1, H), const),
        pl.BlockSpec((1, H), const),
        pl.BlockSpec((H, 128), const),
        pl.BlockSpec((1, 128), const),
        pl.BlockSpec((1, H, H), lambda s: (m(s) // 2, 0, m(s) % 2)),
        pl.BlockSpec((1, 1, H), lambda s: (m(s) // 2, 0, m(s) % 2)),
        pl.BlockSpec((1, H, H), lambda s: (m(s) // 2, m(s) % 2, 0)),
        pl.BlockSpec((1, 1, H), lambda s: (m(s) // 2, 0, 0)),
        pl.BlockSpec((S, H), const),
        pl.BlockSpec((H, VB2), lambda s: (0, j(s))),
        pl.BlockSpec((1, VB2), lambda s: (0, j(s))),
    ]
    return pl.pallas_call(
        _megab_body,
        grid=(N_STEPS,),
        in_specs=specs,
        out_specs=(pl.BlockSpec((S, H), const),
                   pl.BlockSpec((S, VB2), lambda s: (0, j(s)))),
        out_shape=(jax.ShapeDtypeStruct((S, H), jnp.float32),
                   jax.ShapeDtypeStruct((S, V), jnp.float32)),
        scratch_shapes=[
            pltpu.VMEM((S, H), jnp.float32),
            pltpu.VMEM((S, 128), jnp.float32),
        ],
    )(xn, y0, qkv_Ws, qkv_Ws, qkv_Ws, qkv_bs, qkv_bs, qkv_bs, out_W,
      n2g, n2b, gw_pad, gb_pad, w1, b1, w2, b2, x0, hw0, hb0)


VB = 2048


def _head_body(x_ref, w_ref, b_ref, out_ref):
    out_ref[...] = _dot(x_ref[...], w_ref[...]) + b_ref[...]


def _head(x, hw, hb):
    nvb = pl.cdiv(V, VB)
    return pl.pallas_call(
        _head_body,
        grid=(nvb,),
        in_specs=[
            pl.BlockSpec((S, H), lambda j: (0, 0)),
            pl.BlockSpec((H, VB), lambda j: (0, j)),
            pl.BlockSpec((1, VB), lambda j: (0, j)),
        ],
        out_specs=pl.BlockSpec((S, VB), lambda j: (0, j)),
        out_shape=jax.ShapeDtypeStruct((S, V), jnp.float32),
    )(x, hw, hb)


def kernel(hidden_states, token_embeddings, proj_W, proj_b, qkv_W, qkv_b,
           attn_out_W, attn_out_b, norm1_g, norm1_b, norm2_g, norm2_b,
           gate_W, gate_b, w1, b1, w2, b2, head_W, head_b):
    hs = hidden_states.reshape(S, H)

    def parts(i):
        gw_pad = jnp.pad(gate_W[i], ((0, 0), (0, 128 - E)))
        gb_pad = jnp.pad(gate_b[i], (0, 128 - E)).reshape(1, 128)
        qkv_Ws = qkv_W[i].reshape(H, 3 * NH, DH).transpose(1, 0, 2)
        qkv_bs = qkv_b[i].reshape(3 * NH, 1, DH)
        return gw_pad, gb_pad, qkv_Ws, qkv_bs

    gw0, gb0, qw0, qb0 = parts(0)
    xn0, y00 = _prologue(hs, token_embeddings[0, 0],
                         proj_W[0, :H], proj_W[0, H:],
                         proj_b[0].reshape(1, H),
                         norm1_g[0].reshape(1, H), norm1_b[0].reshape(1, H),
                         attn_out_b[0].reshape(1, H))
    y0m = _attention(xn0, y00, qw0, qb0, attn_out_W[0])
    z0 = _moe(y0m, norm2_g[0].reshape(1, H), norm2_b[0].reshape(1, H),
              gw0, gb0, w1[0], b1[0].reshape(E, 1, FF),
              w2[0], b2[0].reshape(E, 1, H))

    gw1, gb1, qw1, qb1 = parts(1)
    xn1, y01 = _prologue(hs, token_embeddings[1, 0],
                         proj_W[1, :H], proj_W[1, H:],
                         proj_b[1].reshape(1, H),
                         norm1_g[1].reshape(1, H), norm1_b[1].reshape(1, H),
                         attn_out_b[1].reshape(1, H))
    z1, logits0 = _megab(xn1, y01, qw1, qb1, attn_out_W[1],
                         norm2_g[1].reshape(1, H), norm2_b[1].reshape(1, H),
                         gw1, gb1, w1[1], b1[1].reshape(E, 1, FF),
                         w2[1], b2[1].reshape(E, 1, H),
                         z0, head_W[0], head_b[0].reshape(1, V))

    logits1 = _head(z1, head_W[1], head_b[1].reshape(1, V))
    mtp_logits = jnp.stack([logits0, logits1])[:, None]
    return mtp_logits, jnp.zeros((), jnp.float32)


# speedup vs baseline: 1.0125x; 1.0125x over previous
import math

import jax
import jax.numpy as jnp
from jax.experimental import pallas as pl
import jax.experimental.pallas.tpu as pltpu

H = 768
V = 32000
NH = 12
DH = H // NH
E = 8
FF = 1536
S = 2048
EPS = 1e-5


def _ln(x, g=None, b=None):
    m = jnp.mean(x, axis=-1, keepdims=True)
    v = jnp.mean(x * x, axis=-1, keepdims=True) - m * m
    y = (x - m) * jax.lax.rsqrt(v + EPS)
    if g is not None:
        y = y * g + b
    return y


def _dot(a, b):
    return jnp.dot(a.astype(jnp.bfloat16), b.astype(jnp.bfloat16),
                   preferred_element_type=jnp.float32)


def _pre_body(hs_ref, te_ref, pw1_ref, pw2_ref, pb_ref,
              n1g_ref, n1b_ref, ob_ref, xn_ref, y_ref):
    c0 = (_dot(_ln(hs_ref[...]), pw1_ref[...])
          + _dot(_ln(te_ref[...]), pw2_ref[...]) + pb_ref[...])
    xn_ref[...] = _ln(c0, n1g_ref[...], n1b_ref[...]).astype(jnp.bfloat16)
    y_ref[...] = c0 + ob_ref[...]


def _prologue(hs, te, pw1, pw2, pb, n1g, n1b, ob):
    return pl.pallas_call(
        _pre_body,
        out_shape=(jax.ShapeDtypeStruct((S, H), jnp.bfloat16),
                   jax.ShapeDtypeStruct((S, H), jnp.float32)),
    )(hs, te, pw1, pw2, pb, n1g, n1b, ob)


def _attn_body(xn_ref, y0_ref, wq_ref, wk_ref, wv_ref,
               bq_ref, bk_ref, bv_ref, wo_ref, out_ref):
    h = pl.program_id(0)
    xn = xn_ref[...]
    q = _dot(xn, wq_ref[0]) + bq_ref[0]
    k = _dot(xn, wk_ref[0]) + bk_ref[0]
    v = _dot(xn, wv_ref[0]) + bv_ref[0]
    sc = jax.lax.dot_general(q.astype(jnp.bfloat16), k.astype(jnp.bfloat16),
                             (((1,), (1,)), ((), ())),
                             preferred_element_type=jnp.float32)
    sc = sc * (1.0 / math.sqrt(DH))
    sc = sc - jnp.max(sc, axis=-1, keepdims=True)
    p = jnp.exp(sc)
    r = jnp.sum(p, axis=-1, keepdims=True)
    o = _dot(p, v) / r
    contrib = _dot(o, wo_ref[...])

    @pl.when(h == 0)
    def _():
        out_ref[...] = y0_ref[...] + contrib

    @pl.when(h > 0)
    def _():
        out_ref[...] += contrib


def _attention(xn, y0, qkv_Ws, qkv_bs, out_W):
    const = lambda h: (0, 0)
    specs = [
        pl.BlockSpec((S, H), const),
        pl.BlockSpec((S, H), const),
        pl.BlockSpec((1, H, DH), lambda h: (h, 0, 0)),
        pl.BlockSpec((1, H, DH), lambda h: (NH + h, 0, 0)),
        pl.BlockSpec((1, H, DH), lambda h: (2 * NH + h, 0, 0)),
        pl.BlockSpec((1, 1, DH), lambda h: (h, 0, 0)),
        pl.BlockSpec((1, 1, DH), lambda h: (NH + h, 0, 0)),
        pl.BlockSpec((1, 1, DH), lambda h: (2 * NH + h, 0, 0)),
        pl.BlockSpec((DH, H), lambda h: (h, 0)),
    ]
    return pl.pallas_call(
        _attn_body,
        grid=(NH,),
        in_specs=specs,
        out_specs=pl.BlockSpec((S, H), const),
        out_shape=jax.ShapeDtypeStruct((S, H), jnp.float32),
    )(xn, y0, qkv_Ws, qkv_Ws, qkv_Ws, qkv_bs, qkv_bs, qkv_bs, out_W)


def _moe_body(y_ref, g_ref, b_ref, gw_ref, gb_ref,
              w1_ref, b1_ref, w2_ref, b2_ref, out_ref, x2_s, wv_s):
    e = pl.program_id(0)

    @pl.when(e == 0)
    def _gate():
        x2 = _ln(y_ref[...], g_ref[...], b_ref[...])
        x2_s[...] = x2
        logits = _dot(x2, gw_ref[...]) + gb_ref[...]
        lane = jax.lax.broadcasted_iota(jnp.int32, logits.shape, 1)
        logits = jnp.where(lane < E, logits, -1e30)
        logits = logits - jnp.max(logits, axis=-1, keepdims=True)
        pexp = jnp.exp(logits)
        probs = pexp / jnp.sum(pexp, axis=-1, keepdims=True)
        m1 = jnp.max(probs, axis=-1, keepdims=True)
        m2 = jnp.max(jnp.where(probs == m1, -1.0, probs),
                     axis=-1, keepdims=True)
        wv_s[...] = jnp.where(probs >= m2, probs, 0.0) / (m1 + m2)

    x2 = x2_s[...]
    lane = jax.lax.broadcasted_iota(jnp.int32, (S, 128), 1)
    onehot = (lane == e).astype(jnp.float32)
    we = jnp.sum(wv_s[...] * onehot, axis=-1, keepdims=True)
    hmat = jnp.maximum(_dot(x2, w1_ref[0]) + b1_ref[0], 0.0)
    contrib = (_dot(hmat, w2_ref[0]) + b2_ref[0]) * we

    @pl.when(e == 0)
    def _():
        out_ref[...] = y_ref[...] + contrib

    @pl.when(e > 0)
    def _():
        out_ref[...] += contrib


def _moe(y, n2g, n2b, gw_pad, gb_pad, w1, b1, w2, b2):
    const = lambda e: (0, 0)
    specs = [
        pl.BlockSpec((S, H), const),
        pl.BlockSpec((1, H), const),
        pl.BlockSpec((1, H), const),
        pl.BlockSpec((H, 128), const),
        pl.BlockSpec((1, 128), const),
        pl.BlockSpec((1, H, FF), lambda e: (e, 0, 0)),
        pl.BlockSpec((1, 1, FF), lambda e: (e, 0, 0)),
        pl.BlockSpec((1, FF, H), lambda e: (e, 0, 0)),
        pl.BlockSpec((1, 1, H), lambda e: (e, 0, 0)),
    ]
    return pl.pallas_call(
        _moe_body,
        grid=(E,),
        in_specs=specs,
        out_specs=pl.BlockSpec((S, H), const),
        out_shape=jax.ShapeDtypeStruct((S, H), jnp.float32),
        scratch_shapes=[
            pltpu.VMEM((S, H), jnp.float32),
            pltpu.VMEM((S, 128), jnp.float32),
        ],
    )(y, n2g, n2b, gw_pad, gb_pad, w1, b1, w2, b2)


VB2 = 512
N_CH = NH + 2 * E
N_HB = (V + VB2 - 1) // VB2
N_TRIPLE = 3 * N_CH
N_STEPS = N_TRIPLE + (N_HB - 2 * N_CH)
QC = 256
TC = 1024


def _mb_c(s):
    return jnp.minimum(s, N_CH - 1)


def _mb_a(s):
    return jnp.clip(_mb_c(s), 0, NH - 1)


def _mb_m(s):
    return jnp.clip(_mb_c(s) - NH, 0, 2 * E - 1)


def _mb_j(s):
    return jnp.maximum(s - N_CH, 0)


def _megab_body(xn_ref, y0_ref, wq_ref, wk_ref, wv_ref,
                bq_ref, bk_ref, bv_ref, wo_ref,
                n2g_ref, n2b_ref, gw_ref, gb_ref,
                w1_ref, b1_ref, w2_ref, b2_ref,
                x0_ref, hw_ref, hb_ref,
                z_ref, out0_ref, xs, wv_s):
    s = pl.program_id(0)
    is_chain = s < N_CH
    cs = _mb_c(s)

    @pl.when(is_chain & (cs < NH))
    def _att():
        xn = xn_ref[...]
        q = _dot(xn, wq_ref[0]) + bq_ref[0]
        k = _dot(xn, wk_ref[0]) + bk_ref[0]
        v = _dot(xn, wv_ref[0]) + bv_ref[0]
        kb = k.astype(jnp.bfloat16)
        for i in range(S // QC):
            qc = q[i * QC:(i + 1) * QC, :]
            sc = jax.lax.dot_general(qc.astype(jnp.bfloat16), kb,
                                     (((1,), (1,)), ((), ())),
                                     preferred_element_type=jnp.float32)
            sc = sc * (1.0 / math.sqrt(DH))
            sc = sc - jnp.max(sc, axis=-1, keepdims=True)
            p = jnp.exp(sc)
            r = jnp.sum(p, axis=-1, keepdims=True)
            o = _dot(p, v) / r
            contrib = _dot(o, wo_ref[...])

            @pl.when(cs == 0)
            def _():
                z_ref[i * QC:(i + 1) * QC, :] = (
                    y0_ref[i * QC:(i + 1) * QC, :] + contrib)

            @pl.when(cs > 0)
            def _():
                z_ref[i * QC:(i + 1) * QC, :] += contrib

    @pl.when(is_chain & (cs >= NH))
    def _moe():
        mm = cs - NH
        e = mm // 2
        f = mm % 2

        @pl.when(mm == 0)
        def _gate():
            x2 = _ln(z_ref[...], n2g_ref[...], n2b_ref[...])
            xs[...] = x2
            logits = _dot(x2, gw_ref[...]) + gb_ref[...]
            lane = jax.lax.broadcasted_iota(jnp.int32, logits.shape, 1)
            logits = jnp.where(lane < E, logits, -1e30)
            logits = logits - jnp.max(logits, axis=-1, keepdims=True)
            pexp = jnp.exp(logits)
            probs = pexp / jnp.sum(pexp, axis=-1, keepdims=True)
            m1 = jnp.max(probs, axis=-1, keepdims=True)
            m2 = jnp.max(jnp.where(probs == m1, -1.0, probs),
                         axis=-1, keepdims=True)
            wv_s[...] = jnp.where(probs >= m2, probs, 0.0) / (m1 + m2)

        lane = jax.lax.broadcasted_iota(jnp.int32, (TC, 128), 1)
        fb2 = (f == 0).astype(jnp.float32) * b2_ref[0]
        for t in range(S // TC):
            sl = slice(t * TC, (t + 1) * TC)
            x2 = xs[sl, :]
            onehot = (lane == e).astype(jnp.float32)
            we = jnp.sum(wv_s[sl, :] * onehot, axis=-1, keepdims=True)
            hmat = jnp.maximum(_dot(x2, w1_ref[0]) + b1_ref[0], 0.0)
            eo = _dot(hmat, w2_ref[0]) + fb2
            z_ref[sl, :] += eo * we

    @pl.when(jnp.logical_not(is_chain))
    def _head():
        out0_ref[...] = _dot(x0_ref[...], hw_ref[...]) + hb_ref[...]


def _megab(xn, y0, qkv_Ws, qkv_bs, out_W, n2g, n2b, gw_pad, gb_pad,
           w1, b1, w2, b2, x0, hw0, hb0):
    const = lambda s: (0, 0)
    a, m, j = _mb_a, _mb_m, _mb_j
    specs = [
        pl.BlockSpec((S, H), const),
        pl.BlockSpec((S, H), const),
        pl.BlockSpec((1, H, DH), lambda s: (a(s), 0, 0)),
        pl.BlockSpec((1, H, DH), lambda s: (NH + a(s), 0, 0)),
        pl.BlockSpec((1, H, DH), lambda s: (2 * NH + a(s), 0, 0)),
        pl.BlockSpec((1, 1, DH), lambda s: (a(s), 0, 0)),
        pl.BlockSpec((1, 1, DH), lambda s: (NH + a(s), 0, 0)),
        pl.BlockSpec((1, 1, DH), lambda s: (2 * NH + a(s), 0, 0)),
        pl.BlockSpec((DH, H), lambda s: (a(s), 0)),
        pl.BlockSpec((1, H), const),
        pl.BlockSpec((1, H), const),
        pl.BlockSpec((H, 128), const),
        pl.BlockSpec((1, 128), const),
        pl.BlockSpec((1, H, H), lambda s: (m(s) // 2, 0, m(s) % 2)),
        pl.BlockSpec((1, 1, H), lambda s: (m(s) // 2, 0, m(s) % 2)),
        pl.BlockSpec((1, H, H), lambda s: (m(s) // 2, m(s) % 2, 0)),
        pl.BlockSpec((1, 1, H), lambda s: (m(s) // 2, 0, 0)),
        pl.BlockSpec((S, H), const),
        pl.BlockSpec((H, VB2), lambda s: (0, j(s))),
        pl.BlockSpec((1, VB2), lambda s: (0, j(s))),
    ]
    return pl.pallas_call(
        _megab_body,
        grid=(N_STEPS,),
        in_specs=specs,
        out_specs=(pl.BlockSpec((S, H), const),
                   pl.BlockSpec((S, VB2), lambda s: (0, j(s)))),
        out_shape=(jax.ShapeDtypeStruct((S, H), jnp.float32),
                   jax.ShapeDtypeStruct((S, V), jnp.float32)),
        scratch_shapes=[
            pltpu.VMEM((S, H), jnp.float32),
            pltpu.VMEM((S, 128), jnp.float32),
        ],
    )(xn, y0, qkv_Ws, qkv_Ws, qkv_Ws, qkv_bs, qkv_bs, qkv_bs, out_W,
      n2g, n2b, gw_pad, gb_pad, w1, b1, w2, b2, x0, hw0, hb0)


VB = 2048


def _head_body(x_ref, w_ref, b_ref, out_ref):
    out_ref[...] = _dot(x_ref[...], w_ref[...]) + b_ref[...]


def _head(x, hw, hb):
    nvb = pl.cdiv(V, VB)
    return pl.pallas_call(
        _head_body,
        grid=(nvb,),
        in_specs=[
            pl.BlockSpec((S, H), lambda j: (0, 0)),
            pl.BlockSpec((H, VB), lambda j: (0, j)),
            pl.BlockSpec((1, VB), lambda j: (0, j)),
        ],
        out_specs=pl.BlockSpec((S, VB), lambda j: (0, j)),
        out_shape=jax.ShapeDtypeStruct((S, V), jnp.float32),
    )(x, hw, hb)


def kernel(hidden_states, token_embeddings, proj_W, proj_b, qkv_W, qkv_b,
           attn_out_W, attn_out_b, norm1_g, norm1_b, norm2_g, norm2_b,
           gate_W, gate_b, w1, b1, w2, b2, head_W, head_b):
    hs = hidden_states.reshape(S, H)

    def parts(i):
        gw_pad = jnp.pad(gate_W[i], ((0, 0), (0, 128 - E)))
        gb_pad = jnp.pad(gate_b[i], (0, 128 - E)).reshape(1, 128)
        qkv_Ws = qkv_W[i].reshape(H, 3 * NH, DH).transpose(1, 0, 2)
        qkv_bs = qkv_b[i].reshape(3 * NH, 1, DH)
        return gw_pad, gb_pad, qkv_Ws, qkv_bs

    gw0, gb0, qw0, qb0 = parts(0)
    xn0, y00 = _prologue(hs, token_embeddings[0, 0],
                         proj_W[0, :H], proj_W[0, H:],
                         proj_b[0].reshape(1, H),
                         norm1_g[0].reshape(1, H), norm1_b[0].reshape(1, H),
                         attn_out_b[0].reshape(1, H))
    y0m = _attention(xn0, y00, qw0, qb0, attn_out_W[0])
    z0 = _moe(y0m, norm2_g[0].reshape(1, H), norm2_b[0].reshape(1, H),
              gw0, gb0, w1[0], b1[0].reshape(E, 1, FF),
              w2[0], b2[0].reshape(E, 1, H))

    gw1, gb1, qw1, qb1 = parts(1)
    xn1, y01 = _prologue(hs, token_embeddings[1, 0],
                         proj_W[1, :H], proj_W[1, H:],
                         proj_b[1].reshape(1, H),
                         norm1_g[1].reshape(1, H), norm1_b[1].reshape(1, H),
                         attn_out_b[1].reshape(1, H))
    z1, logits0 = _megab(xn1, y01, qw1, qb1, attn_out_W[1],
                         norm2_g[1].reshape(1, H), norm2_b[1].reshape(1, H),
                         gw1, gb1, w1[1], b1[1].reshape(E, 1, FF),
                         w2[1], b2[1].reshape(E, 1, H),
                         z0, head_W[0], head_b[0].reshape(1, V))

    logits1 = _head(z1, head_W[1], head_b[1].reshape(1, V))
    mtp_logits = jnp.stack([logits0, logits1])[:, None]
    return mtp_logits, jnp.zeros((), jnp.float32)


# speedup vs baseline: 1.0169x; 1.0043x over previous
import math

import jax
import jax.numpy as jnp
from jax.experimental import pallas as pl
import jax.experimental.pallas.tpu as pltpu

H = 768
V = 32000
NH = 12
DH = H // NH
E = 8
FF = 1536
S = 2048
EPS = 1e-5


def _ln(x, g=None, b=None):
    m = jnp.mean(x, axis=-1, keepdims=True)
    v = jnp.mean(x * x, axis=-1, keepdims=True) - m * m
    y = (x - m) * jax.lax.rsqrt(v + EPS)
    if g is not None:
        y = y * g + b
    return y


def _dot(a, b):
    return jnp.dot(a.astype(jnp.bfloat16), b.astype(jnp.bfloat16),
                   preferred_element_type=jnp.float32)


def _pre_body(hs_ref, te_ref, pw1_ref, pw2_ref, pb_ref,
              n1g_ref, n1b_ref, ob_ref, xn_ref, y_ref):
    c0 = (_dot(_ln(hs_ref[...]), pw1_ref[...])
          + _dot(_ln(te_ref[...]), pw2_ref[...]) + pb_ref[...])
    xn_ref[...] = _ln(c0, n1g_ref[...], n1b_ref[...]).astype(jnp.bfloat16)
    y_ref[...] = c0 + ob_ref[...]


def _prologue(hs, te, pw1, pw2, pb, n1g, n1b, ob):
    return pl.pallas_call(
        _pre_body,
        out_shape=(jax.ShapeDtypeStruct((S, H), jnp.bfloat16),
                   jax.ShapeDtypeStruct((S, H), jnp.float32)),
    )(hs, te, pw1, pw2, pb, n1g, n1b, ob)


def _attn_body(xn_ref, y0_ref, wq_ref, wk_ref, wv_ref,
               bq_ref, bk_ref, bv_ref, wo_ref, out_ref):
    h = pl.program_id(0)
    xn = xn_ref[...]
    q = _dot(xn, wq_ref[0]) + bq_ref[0]
    k = _dot(xn, wk_ref[0]) + bk_ref[0]
    v = (_dot(xn, wv_ref[0]) + bv_ref[0]).astype(jnp.bfloat16)
    sc = jax.lax.dot_general(q.astype(jnp.bfloat16), k.astype(jnp.bfloat16),
                             (((1,), (1,)), ((), ())),
                             preferred_element_type=jnp.float32)
    sc = sc * (1.0 / math.sqrt(DH))
    sc = sc - jnp.max(sc, axis=-1, keepdims=True)
    p = jnp.exp(sc.astype(jnp.bfloat16))
    r = jnp.sum(p.astype(jnp.float32), axis=-1, keepdims=True)
    o = jnp.dot(p, v, preferred_element_type=jnp.float32) / r
    contrib = _dot(o, wo_ref[...])

    @pl.when(h == 0)
    def _():
        out_ref[...] = y0_ref[...] + contrib

    @pl.when(h > 0)
    def _():
        out_ref[...] += contrib


def _attention(xn, y0, qkv_Ws, qkv_bs, out_W):
    const = lambda h: (0, 0)
    specs = [
        pl.BlockSpec((S, H), const),
        pl.BlockSpec((S, H), const),
        pl.BlockSpec((1, H, DH), lambda h: (h, 0, 0)),
        pl.BlockSpec((1, H, DH), lambda h: (NH + h, 0, 0)),
        pl.BlockSpec((1, H, DH), lambda h: (2 * NH + h, 0, 0)),
        pl.BlockSpec((1, 1, DH), lambda h: (h, 0, 0)),
        pl.BlockSpec((1, 1, DH), lambda h: (NH + h, 0, 0)),
        pl.BlockSpec((1, 1, DH), lambda h: (2 * NH + h, 0, 0)),
        pl.BlockSpec((DH, H), lambda h: (h, 0)),
    ]
    return pl.pallas_call(
        _attn_body,
        grid=(NH,),
        in_specs=specs,
        out_specs=pl.BlockSpec((S, H), const),
        out_shape=jax.ShapeDtypeStruct((S, H), jnp.float32),
    )(xn, y0, qkv_Ws, qkv_Ws, qkv_Ws, qkv_bs, qkv_bs, qkv_bs, out_W)


def _moe_body(y_ref, g_ref, b_ref, gw_ref, gb_ref,
              w1_ref, b1_ref, w2_ref, b2_ref, out_ref, x2_s, wv_s):
    e = pl.program_id(0)

    @pl.when(e == 0)
    def _gate():
        x2 = _ln(y_ref[...], g_ref[...], b_ref[...])
        x2_s[...] = x2.astype(jnp.bfloat16)
        logits = _dot(x2, gw_ref[...]) + gb_ref[...]
        lane = jax.lax.broadcasted_iota(jnp.int32, logits.shape, 1)
        logits = jnp.where(lane < E, logits, -1e30)
        logits = logits - jnp.max(logits, axis=-1, keepdims=True)
        pexp = jnp.exp(logits)
        probs = pexp / jnp.sum(pexp, axis=-1, keepdims=True)
        m1 = jnp.max(probs, axis=-1, keepdims=True)
        m2 = jnp.max(jnp.where(probs == m1, -1.0, probs),
                     axis=-1, keepdims=True)
        wv_s[...] = jnp.where(probs >= m2, probs, 0.0) / (m1 + m2)

    x2 = x2_s[...]
    lane = jax.lax.broadcasted_iota(jnp.int32, (S, 128), 1)
    onehot = (lane == e).astype(jnp.float32)
    we = jnp.sum(wv_s[...] * onehot, axis=-1, keepdims=True)
    hmat = jnp.maximum(
        jnp.dot(x2, w1_ref[0].astype(jnp.bfloat16),
                preferred_element_type=jnp.float32) + b1_ref[0],
        0.0).astype(jnp.bfloat16)
    contrib = (jnp.dot(hmat, w2_ref[0].astype(jnp.bfloat16),
                       preferred_element_type=jnp.float32)
               + b2_ref[0]) * we

    @pl.when(e == 0)
    def _():
        out_ref[...] = y_ref[...] + contrib

    @pl.when(e > 0)
    def _():
        out_ref[...] += contrib


def _moe(y, n2g, n2b, gw_pad, gb_pad, w1, b1, w2, b2):
    const = lambda e: (0, 0)
    specs = [
        pl.BlockSpec((S, H), const),
        pl.BlockSpec((1, H), const),
        pl.BlockSpec((1, H), const),
        pl.BlockSpec((H, 128), const),
        pl.BlockSpec((1, 128), const),
        pl.BlockSpec((1, H, FF), lambda e: (e, 0, 0)),
        pl.BlockSpec((1, 1, FF), lambda e: (e, 0, 0)),
        pl.BlockSpec((1, FF, H), lambda e: (e, 0, 0)),
        pl.BlockSpec((1, 1, H), lambda e: (e, 0, 0)),
    ]
    return pl.pallas_call(
        _moe_body,
        grid=(E,),
        in_specs=specs,
        out_specs=pl.BlockSpec((S, H), const),
        out_shape=jax.ShapeDtypeStruct((S, H), jnp.float32),
        scratch_shapes=[
            pltpu.VMEM((S, H), jnp.bfloat16),
            pltpu.VMEM((S, 128), jnp.float32),
        ],
    )(y, n2g, n2b, gw_pad, gb_pad, w1, b1, w2, b2)


VB = 1280
NVB = V // VB


def _head_body(x_ref, w_ref, b_ref, out_hbm, buf, sems):
    j = pl.program_id(0)
    slot = j % 2

    @pl.when(j >= 2)
    def _():
        pltpu.make_async_copy(
            buf.at[slot], out_hbm.at[:, pl.ds((j - 2) * VB, VB)],
            sems.at[slot]).wait()

    buf[slot] = _dot(x_ref[...], w_ref[...]) + b_ref[...]
    pltpu.make_async_copy(
        buf.at[slot], out_hbm.at[:, pl.ds(j * VB, VB)], sems.at[slot]).start()

    @pl.when(j == NVB - 1)
    def _():
        pltpu.make_async_copy(
            buf.at[1 - slot], out_hbm.at[:, pl.ds((j - 1) * VB, VB)],
            sems.at[1 - slot]).wait()
        pltpu.make_async_copy(
            buf.at[slot], out_hbm.at[:, pl.ds(j * VB, VB)],
            sems.at[slot]).wait()


def _head(x, hw, hb):
    return pl.pallas_call(
        _head_body,
        grid=(NVB,),
        in_specs=[
            pl.BlockSpec((S, H), lambda j: (0, 0)),
            pl.BlockSpec((H, VB), lambda j: (0, j)),
            pl.BlockSpec((1, VB), lambda j: (0, j)),
        ],
        out_specs=pl.BlockSpec(memory_space=pl.ANY),
        out_shape=jax.ShapeDtypeStruct((S, V), jnp.float32),
        scratch_shapes=[
            pltpu.VMEM((2, S, VB), jnp.float32),
            pltpu.SemaphoreType.DMA((2,)),
        ],
    )(x, hw, hb)


def kernel(hidden_states, token_embeddings, proj_W, proj_b, qkv_W, qkv_b,
           attn_out_W, attn_out_b, norm1_g, norm1_b, norm2_g, norm2_b,
           gate_W, gate_b, w1, b1, w2, b2, head_W, head_b):
    nmtp = proj_W.shape[0]
    hs = hidden_states.reshape(S, H)
    outs = []
    for i in range(nmtp):
        gw_pad = jnp.pad(gate_W[i], ((0, 0), (0, 128 - E)))
        gb_pad = jnp.pad(gate_b[i], (0, 128 - E)).reshape(1, 128)
        qkv_Ws = qkv_W[i].reshape(H, 3 * NH, DH).transpose(1, 0, 2)
        qkv_bs = qkv_b[i].reshape(3 * NH, 1, DH)
        xn, y0 = _prologue(hs, token_embeddings[i, 0],
                           proj_W[i, :H], proj_W[i, H:],
                           proj_b[i].reshape(1, H),
                           norm1_g[i].reshape(1, H), norm1_b[i].reshape(1, H),
                           attn_out_b[i].reshape(1, H))
        y = _attention(xn, y0, qkv_Ws, qkv_bs, attn_out_W[i])
        z = _moe(y, norm2_g[i].reshape(1, H), norm2_b[i].reshape(1, H),
                 gw_pad, gb_pad, w1[i], b1[i].reshape(E, 1, FF),
                 w2[i], b2[i].reshape(E, 1, H))
        outs.append(_head(z, head_W[i], head_b[i].reshape(1, V)))
    mtp_logits = jnp.stack(outs)[:, None]
    return mtp_logits, jnp.zeros((), jnp.float32)


# speedup vs baseline: 1.0553x; 1.0378x over previous
import math

import jax
import jax.numpy as jnp
from jax.experimental import pallas as pl
import jax.experimental.pallas.tpu as pltpu

H = 768
V = 32000
NH = 12
DH = H // NH
E = 8
FF = 1536
S = 2048
EPS = 1e-5


def _ln(x, g=None, b=None):
    m = jnp.mean(x, axis=-1, keepdims=True)
    v = jnp.mean(x * x, axis=-1, keepdims=True) - m * m
    y = (x - m) * jax.lax.rsqrt(v + EPS)
    if g is not None:
        y = y * g + b
    return y


def _dot(a, b):
    return jnp.dot(a.astype(jnp.bfloat16), b.astype(jnp.bfloat16),
                   preferred_element_type=jnp.float32)


def _pre_body(hs_ref, te_ref, pw1_ref, pw2_ref, pb_ref,
              n1g_ref, n1b_ref, ob_ref, xn_ref, y_ref):
    c0 = (_dot(_ln(hs_ref[...]), pw1_ref[...])
          + _dot(_ln(te_ref[...]), pw2_ref[...]) + pb_ref[...])
    xn_ref[...] = _ln(c0, n1g_ref[...], n1b_ref[...]).astype(jnp.bfloat16)
    y_ref[...] = c0 + ob_ref[...]


def _prologue(hs, te, pw1, pw2, pb, n1g, n1b, ob):
    return pl.pallas_call(
        _pre_body,
        out_shape=(jax.ShapeDtypeStruct((S, H), jnp.bfloat16),
                   jax.ShapeDtypeStruct((S, H), jnp.float32)),
    )(hs, te, pw1, pw2, pb, n1g, n1b, ob)


HPS = 2
AQC = 1024


def _attn_body(xn_ref, y0_ref, wq_ref, wk_ref, wv_ref,
               bq_ref, bk_ref, bv_ref, wo_ref, out_ref):
    g = pl.program_id(0)
    xn = xn_ref[...]
    contribs = []
    for hh in range(HPS):
        q = _dot(xn, wq_ref[hh]) + bq_ref[hh]
        k = _dot(xn, wk_ref[hh]) + bk_ref[hh]
        v = (_dot(xn, wv_ref[hh]) + bv_ref[hh]).astype(jnp.bfloat16)
        kb = k.astype(jnp.bfloat16)
        wo_h = wo_ref[hh * DH:(hh + 1) * DH, :]
        for c in range(S // AQC):
            qc = q[c * AQC:(c + 1) * AQC, :]
            sc = jax.lax.dot_general(qc.astype(jnp.bfloat16), kb,
                                     (((1,), (1,)), ((), ())),
                                     preferred_element_type=jnp.float32)
            sc = sc * (1.0 / math.sqrt(DH))
            sc = sc - jnp.max(sc, axis=-1, keepdims=True)
            p = jnp.exp(sc.astype(jnp.bfloat16))
            r = jnp.sum(p.astype(jnp.float32), axis=-1, keepdims=True)
            o = jnp.dot(p, v, preferred_element_type=jnp.float32) / r
            contribs.append((c, _dot(o, wo_h)))
    acc = [jnp.zeros((AQC, H), jnp.float32) for _ in range(S // AQC)]
    for c, contrib in contribs:
        acc[c] = acc[c] + contrib
    for c in range(S // AQC):
        sl = slice(c * AQC, (c + 1) * AQC)

        @pl.when(g == 0)
        def _(sl=sl, c=c):
            out_ref[sl, :] = y0_ref[sl, :] + acc[c]

        @pl.when(g > 0)
        def _(sl=sl, c=c):
            out_ref[sl, :] += acc[c]


def _attention(xn, y0, qkv_Ws, qkv_bs, out_W):
    const = lambda h: (0, 0)
    specs = [
        pl.BlockSpec((S, H), const),
        pl.BlockSpec((S, H), const),
        pl.BlockSpec((HPS, H, DH), lambda g: (g, 0, 0)),
        pl.BlockSpec((HPS, H, DH), lambda g: (NH // HPS + g, 0, 0)),
        pl.BlockSpec((HPS, H, DH), lambda g: (2 * NH // HPS + g, 0, 0)),
        pl.BlockSpec((HPS, 1, DH), lambda g: (g, 0, 0)),
        pl.BlockSpec((HPS, 1, DH), lambda g: (NH // HPS + g, 0, 0)),
        pl.BlockSpec((HPS, 1, DH), lambda g: (2 * NH // HPS + g, 0, 0)),
        pl.BlockSpec((HPS * DH, H), lambda g: (g, 0)),
    ]
    return pl.pallas_call(
        _attn_body,
        grid=(NH // HPS,),
        in_specs=specs,
        out_specs=pl.BlockSpec((S, H), const),
        out_shape=jax.ShapeDtypeStruct((S, H), jnp.float32),
    )(xn, y0, qkv_Ws, qkv_Ws, qkv_Ws, qkv_bs, qkv_bs, qkv_bs, out_W)


def _moe_body(y_ref, g_ref, b_ref, gw_ref, gb_ref,
              w1_ref, b1_ref, w2_ref, b2_ref, out_ref, x2_s, wv_s):
    e = pl.program_id(0)

    @pl.when(e == 0)
    def _gate():
        x2 = _ln(y_ref[...], g_ref[...], b_ref[...])
        x2_s[...] = x2.astype(jnp.bfloat16)
        logits = _dot(x2, gw_ref[...]) + gb_ref[...]
        lane = jax.lax.broadcasted_iota(jnp.int32, logits.shape, 1)
        logits = jnp.where(lane < E, logits, -1e30)
        logits = logits - jnp.max(logits, axis=-1, keepdims=True)
        pexp = jnp.exp(logits)
        probs = pexp / jnp.sum(pexp, axis=-1, keepdims=True)
        m1 = jnp.max(probs, axis=-1, keepdims=True)
        m2 = jnp.max(jnp.where(probs == m1, -1.0, probs),
                     axis=-1, keepdims=True)
        wv_s[...] = jnp.where(probs >= m2, probs, 0.0) / (m1 + m2)

    x2 = x2_s[...]
    lane = jax.lax.broadcasted_iota(jnp.int32, (S, 128), 1)
    onehot = (lane == e).astype(jnp.float32)
    we = jnp.sum(wv_s[...] * onehot, axis=-1, keepdims=True)
    hmat = jnp.maximum(
        jnp.dot(x2, w1_ref[0].astype(jnp.bfloat16),
                preferred_element_type=jnp.float32) + b1_ref[0],
        0.0).astype(jnp.bfloat16)
    contrib = (jnp.dot(hmat, w2_ref[0].astype(jnp.bfloat16),
                       preferred_element_type=jnp.float32)
               + b2_ref[0]) * we

    @pl.when(e == 0)
    def _():
        out_ref[...] = y_ref[...] + contrib

    @pl.when(e > 0)
    def _():
        out_ref[...] += contrib


def _moe(y, n2g, n2b, gw_pad, gb_pad, w1, b1, w2, b2):
    const = lambda e: (0, 0)
    specs = [
        pl.BlockSpec((S, H), const),
        pl.BlockSpec((1, H), const),
        pl.BlockSpec((1, H), const),
        pl.BlockSpec((H, 128), const),
        pl.BlockSpec((1, 128), const),
        pl.BlockSpec((1, H, FF), lambda e: (e, 0, 0)),
        pl.BlockSpec((1, 1, FF), lambda e: (e, 0, 0)),
        pl.BlockSpec((1, FF, H), lambda e: (e, 0, 0)),
        pl.BlockSpec((1, 1, H), lambda e: (e, 0, 0)),
    ]
    return pl.pallas_call(
        _moe_body,
        grid=(E,),
        in_specs=specs,
        out_specs=pl.BlockSpec((S, H), const),
        out_shape=jax.ShapeDtypeStruct((S, H), jnp.float32),
        scratch_shapes=[
            pltpu.VMEM((S, H), jnp.bfloat16),
            pltpu.VMEM((S, 128), jnp.float32),
        ],
    )(y, n2g, n2b, gw_pad, gb_pad, w1, b1, w2, b2)


VB = 1280
NVB = V // VB


def _head_body(x_ref, w_ref, b_ref, out_hbm, buf, sems):
    j = pl.program_id(0)
    slot = j % 2

    @pl.when(j >= 2)
    def _():
        pltpu.make_async_copy(
            buf.at[slot], out_hbm.at[:, pl.ds((j - 2) * VB, VB)],
            sems.at[slot]).wait()

    buf[slot] = _dot(x_ref[...], w_ref[...]) + b_ref[...]
    pltpu.make_async_copy(
        buf.at[slot], out_hbm.at[:, pl.ds(j * VB, VB)], sems.at[slot]).start()

    @pl.when(j == NVB - 1)
    def _():
        pltpu.make_async_copy(
            buf.at[1 - slot], out_hbm.at[:, pl.ds((j - 1) * VB, VB)],
            sems.at[1 - slot]).wait()
        pltpu.make_async_copy(
            buf.at[slot], out_hbm.at[:, pl.ds(j * VB, VB)],
            sems.at[slot]).wait()


def _head(x, hw, hb):
    return pl.pallas_call(
        _head_body,
        grid=(NVB,),
        in_specs=[
            pl.BlockSpec((S, H), lambda j: (0, 0)),
            pl.BlockSpec((H, VB), lambda j: (0, j)),
            pl.BlockSpec((1, VB), lambda j: (0, j)),
        ],
        out_specs=pl.BlockSpec(memory_space=pl.ANY),
        out_shape=jax.ShapeDtypeStruct((S, V), jnp.float32),
        scratch_shapes=[
            pltpu.VMEM((2, S, VB), jnp.float32),
            pltpu.SemaphoreType.DMA((2,)),
        ],
    )(x, hw, hb)


def kernel(hidden_states, token_embeddings, proj_W, proj_b, qkv_W, qkv_b,
           attn_out_W, attn_out_b, norm1_g, norm1_b, norm2_g, norm2_b,
           gate_W, gate_b, w1, b1, w2, b2, head_W, head_b):
    nmtp = proj_W.shape[0]
    hs = hidden_states.reshape(S, H)
    outs = []
    for i in range(nmtp):
        gw_pad = jnp.pad(gate_W[i], ((0, 0), (0, 128 - E)))
        gb_pad = jnp.pad(gate_b[i], (0, 128 - E)).reshape(1, 128)
        qkv_Ws = qkv_W[i].reshape(H, 3 * NH, DH).transpose(1, 0, 2)
        qkv_bs = qkv_b[i].reshape(3 * NH, 1, DH)
        xn, y0 = _prologue(hs, token_embeddings[i, 0],
                           proj_W[i, :H], proj_W[i, H:],
                           proj_b[i].reshape(1, H),
                           norm1_g[i].reshape(1, H), norm1_b[i].reshape(1, H),
                           attn_out_b[i].reshape(1, H))
        y = _attention(xn, y0, qkv_Ws, qkv_bs, attn_out_W[i])
        z = _moe(y, norm2_g[i].reshape(1, H), norm2_b[i].reshape(1, H),
                 gw_pad, gb_pad, w1[i], b1[i].reshape(E, 1, FF),
                 w2[i], b2[i].reshape(E, 1, H))
        outs.append(_head(z, head_W[i], head_b[i].reshape(1, V)))
    mtp_logits = jnp.stack(outs)[:, None]
    return mtp_logits, jnp.zeros((), jnp.float32)
